# Initial kernel scaffold; baseline (speedup 1.0000x reference)
import jax
import jax.numpy as jnp
from jax.experimental import pallas as pl


def kernel(x, edge_index, edge_type, comp1, bases1, root1, bias1, comp2, bases2, root2, bias2):
    raise NotImplementedError("write your pallas kernel here")


# speedup vs baseline: 25.1153x; 25.1153x over previous
import functools

import jax
import jax.numpy as jnp
from jax import lax
from jax.experimental import pallas as pl
from jax.experimental.pallas import tpu as pltpu
from jax.experimental.pallas import tpu_sc as plsc

N = 10000
E = 320000
C = 128
R = 8
NB = 4
NR = N * R

NCORES = 2
NSUB = 16
EPT = E // (NCORES * NSUB)
EPA = E // NSUB
CHUNK = 80
SBLK = 2000
SCH = SBLK // CHUNK
ROWS_PER_TILE = 640
ACC_ROWS = NSUB * ROWS_PER_TILE
CNT_PER_TILE = 5120
CNT_TOT = NSUB * CNT_PER_TILE


BR = 1000


def _mk_weight(comp_ref, bases_ref, r):
    wr = comp_ref[r, 0] * bases_ref[0]
    for b in range(1, NB):
        wr = wr + comp_ref[r, b] * bases_ref[b]
    return wr


def _tc1_body(x_ref, comp_ref, bases_ref, root_ref, bias_ref, y_ref, base_ref):
    xb = x_ref[...]
    for r in range(R):
        wr = _mk_weight(comp_ref, bases_ref, r)
        y_ref[:, r * C:(r + 1) * C] = jnp.dot(xb, wr, preferred_element_type=jnp.float32)
    base_ref[...] = jnp.dot(xb, root_ref[...], preferred_element_type=jnp.float32) + bias_ref[...]


def _tc2_body(p_ref, b1_ref, comp_ref, bases_ref, root_ref, bias_ref, y_ref, base_ref):
    h = jnp.maximum(p_ref[0] + p_ref[1] + b1_ref[...], 0.0)
    for r in range(R):
        wr = _mk_weight(comp_ref, bases_ref, r)
        y_ref[:, r * C:(r + 1) * C] = jnp.dot(h, wr, preferred_element_type=jnp.float32)
    base_ref[...] = jnp.dot(h, root_ref[...], preferred_element_type=jnp.float32) + bias_ref[...]


def _tc3_body(p_ref, b2_ref, o_ref):
    o_ref[...] = p_ref[0] + p_ref[1] + b2_ref[...]


_W_SPECS = [
    pl.BlockSpec(memory_space=pltpu.SMEM),
    pl.BlockSpec((NB, C, C), lambda i: (0, 0, 0)),
    pl.BlockSpec((C, C), lambda i: (0, 0)),
    pl.BlockSpec((1, C), lambda i: (0, 0)),
]

_Y_OUT = (
    jax.ShapeDtypeStruct((N, R * C), jnp.float32),
    jax.ShapeDtypeStruct((N, C), jnp.float32),
)
_Y_SPECS = (
    pl.BlockSpec((BR, R * C), lambda i: (i, 0)),
    pl.BlockSpec((BR, C), lambda i: (i, 0)),
)


def _tc1(x, comp, bases, root, bias):
    return pl.pallas_call(
        _tc1_body,
        grid=(N // BR,),
        in_specs=[pl.BlockSpec((BR, C), lambda i: (i, 0))] + _W_SPECS,
        out_specs=_Y_SPECS,
        out_shape=_Y_OUT,
    )(x, comp, bases, root, bias)


def _tc2(part, base1, comp, bases, root, bias):
    return pl.pallas_call(
        _tc2_body,
        grid=(N // BR,),
        in_specs=[
            pl.BlockSpec((NCORES, BR, C), lambda i: (0, i, 0)),
            pl.BlockSpec((BR, C), lambda i: (i, 0)),
        ] + _W_SPECS,
        out_specs=_Y_SPECS,
        out_shape=_Y_OUT,
    )(part, base1, comp, bases, root, bias)


def _tc3(part, base2):
    return pl.pallas_call(
        _tc3_body,
        grid=(N // BR,),
        in_specs=[
            pl.BlockSpec((NCORES, BR, C), lambda i: (0, i, 0)),
            pl.BlockSpec((BR, C), lambda i: (i, 0)),
        ],
        out_specs=pl.BlockSpec((BR, C), lambda i: (i, 0)),
        out_shape=jax.ShapeDtypeStruct((N, C), jnp.float32),
    )(part, base2)


_SC_SCRATCH = [
    pltpu.VMEM_SHARED((ACC_ROWS, C), jnp.float32),
    pltpu.VMEM_SHARED((CNT_TOT,), jnp.float32),
    pltpu.VMEM((SBLK,), jnp.int32),
    pltpu.VMEM((SBLK,), jnp.int32),
    pltpu.VMEM((SBLK,), jnp.int32),
    pltpu.VMEM((CHUNK, C), jnp.float32),
    pltpu.VMEM((CHUNK, C), jnp.float32),
    pltpu.VMEM((CHUNK,), jnp.int32),
    pltpu.VMEM((CHUNK,), jnp.int32),
    pltpu.VMEM((CHUNK,), jnp.int32),
    pltpu.VMEM((CHUNK,), jnp.int32),
    pltpu.VMEM((CHUNK,), jnp.float32),
    pltpu.VMEM((CHUNK + 16,), jnp.float32),
    pltpu.VMEM((CHUNK,), jnp.float32),
    pltpu.SemaphoreType.DMA,
    pltpu.SemaphoreType.DMA,
]


@functools.lru_cache(maxsize=None)
def _sc_mesh():
    return plsc.VectorSubcoreMesh(
        core_axis_name="c", subcore_axis_name="s",
        num_cores=NCORES, num_subcores=NSUB)


def _sc_body(compute_counts, *refs):
    if compute_counts:
        (src_hbm, dst_hbm, et_hbm, y_hbm, zrows_hbm, zcnt_hbm,
         part_hbm, cnt_hbm,
         acc_sh, cnt_sh, srcb, dstb, etb, rows0, rows1, iy0, iy1,
         idxd, ic, cbuf, scale, ones, sem0, sem1) = refs
    else:
        (src_hbm, dst_hbm, et_hbm, y_hbm, zrows_hbm, zcnt_hbm, cnt_in_hbm,
         part_hbm,
         acc_sh, cnt_sh, srcb, dstb, etb, rows0, rows1, iy0, iy1,
         idxd, ic, cbuf, scale, ones, sem0, sem1) = refs
    sems = (sem0, sem1)
    rowbufs = (rows0, rows1)
    iybufs = (iy0, iy1)
    cid = lax.axis_index("c")
    sid = lax.axis_index("s")

    for q in range(CHUNK // 16):
        ones[pl.ds(q * 16, 16)] = jnp.ones((16,), jnp.float32)

    pltpu.sync_copy(zrows_hbm, acc_sh.at[pl.ds(sid * ROWS_PER_TILE, ROWS_PER_TILE)])

    if compute_counts:
        pltpu.sync_copy(zcnt_hbm, cnt_sh.at[pl.ds(sid * CNT_PER_TILE, CNT_PER_TILE)])
    else:
        pltpu.sync_copy(cnt_in_hbm.at[pl.ds(sid * CNT_PER_TILE, CNT_PER_TILE)],
                        cnt_sh.at[pl.ds(sid * CNT_PER_TILE, CNT_PER_TILE)])

    plsc.subcore_barrier()

    if compute_counts:
        def asuper(sb, _):
            base = sid * EPA + sb * SBLK
            pltpu.sync_copy(dst_hbm.at[pl.ds(base, SBLK)], dstb)
            pltpu.sync_copy(et_hbm.at[pl.ds(base, SBLK)], etb)

            def cbody(k, _):
                for q in range(CHUNK // 16):
                    sl = pl.ds(k * CHUNK + q * 16, 16)
                    ic[pl.ds(q * 16, 16)] = dstb[sl] * R + etb[sl]
                pltpu.sync_copy(ones, cnt_sh.at[ic], add=True)
                return 0
            lax.fori_loop(0, SCH, cbody, 0)
            return 0
        lax.fori_loop(0, EPA // SBLK, asuper, 0)
        plsc.subcore_barrier()

        @pl.when(cid == 0)
        def _():
            pltpu.sync_copy(cnt_sh.at[pl.ds(sid * CNT_PER_TILE, CNT_PER_TILE)],
                            cnt_hbm.at[pl.ds(sid * CNT_PER_TILE, CNT_PER_TILE)])

    def issue(k, b):
        for q in range(CHUNK // 16):
            sl = pl.ds(k * CHUNK + q * 16, 16)
            iybufs[b][pl.ds(q * 16, 16)] = srcb[sl] * R + etb[sl]
        pltpu.async_copy(y_hbm.at[iybufs[b]], rowbufs[b], sems[b])

    def process(k, b):
        pltpu.make_async_copy(y_hbm.at[iybufs[b]], rowbufs[b], sems[b]).wait()
        for q in range(CHUNK // 16):
            dsl = pl.ds(k * CHUNK + q * 16, 16)
            dv = dstb[dsl]
            idxd[pl.ds(q * 16, 16)] = dv
            ic[pl.ds(q * 16, 16)] = dv * R + etb[dsl]
        pltpu.sync_copy(cnt_sh.at[ic], cbuf)
        for q in range(CHUNK // 16):
            cv = cbuf[pl.ds(q * 16, 16)]
            scale[pl.ds(q * 16, 16)] = 1.0 / jnp.maximum(cv, 1.0)

        rb = rowbufs[b]

        def mrow(j, _):
            scj = scale[pl.ds(j, 16)][0]
            for q in range(C // 16):
                sl = pl.ds(q * 16, 16)
                rb[j, sl] = rb[j, sl] * scj
            return 0
        lax.fori_loop(0, CHUNK, mrow, 0)
        pltpu.sync_copy(rb, acc_sh.at[idxd], add=True)

    def bsuper(sb, _):
        base = sid * EPA + cid * EPT + sb * SBLK
        pltpu.sync_copy(src_hbm.at[pl.ds(base, SBLK)], srcb)
        pltpu.sync_copy(dst_hbm.at[pl.ds(base, SBLK)], dstb)
        pltpu.sync_copy(et_hbm.at[pl.ds(base, SBLK)], etb)
        issue(0, 0)

        def outer(j, _):
            k0 = j * 2
            issue(k0 + 1, 1)
            process(k0, 0)
            issue(k0 + 2, 0)
            process(k0 + 1, 1)
            return 0
        lax.fori_loop(0, (SCH - 1) // 2, outer, 0)
        process(SCH - 1, 0)
        return 0
    lax.fori_loop(0, EPT // SBLK, bsuper, 0)

    plsc.subcore_barrier()
    pltpu.sync_copy(acc_sh.at[pl.ds(sid * ROWS_PER_TILE, ROWS_PER_TILE)],
                    part_hbm.at[pl.ds(cid * ACC_ROWS + sid * ROWS_PER_TILE, ROWS_PER_TILE)])


@functools.lru_cache(maxsize=None)
def _sc_pass(compute_counts):
    if compute_counts:
        out_type = (
            jax.ShapeDtypeStruct((NCORES * ACC_ROWS, C), jnp.float32),
            jax.ShapeDtypeStruct((CNT_TOT,), jnp.float32),
        )
    else:
        out_type = jax.ShapeDtypeStruct((NCORES * ACC_ROWS, C), jnp.float32)
    return pl.kernel(
        functools.partial(_sc_body, compute_counts),
        out_type=out_type,
        mesh=_sc_mesh(),
        scratch_types=_SC_SCRATCH,
    )


def kernel(x, edge_index, edge_type, comp1, bases1, root1, bias1,
           comp2, bases2, root2, bias2):
    src = edge_index[0].astype(jnp.int32)
    dst = edge_index[1].astype(jnp.int32)
    et = edge_type.astype(jnp.int32)
    zrows = jnp.zeros((ROWS_PER_TILE, C), jnp.float32)
    zcnt = jnp.zeros((CNT_PER_TILE,), jnp.float32)

    y1, base1 = _tc1(x, comp1, bases1, root1, bias1.reshape(1, C))
    part1, cnt = _sc_pass(True)(src, dst, et, y1.reshape(NR, C), zrows, zcnt)
    y2, base2 = _tc2(part1.reshape(NCORES, ACC_ROWS, C), base1,
                     comp2, bases2, root2, bias2.reshape(1, C))
    part2 = _sc_pass(False)(src, dst, et, y2.reshape(NR, C), zrows, zcnt, cnt)
    return _tc3(part2.reshape(NCORES, ACC_ROWS, C), base2)


# speedup vs baseline: 29.5941x; 1.1783x over previous
import functools

import jax
import jax.numpy as jnp
from jax import lax
from jax.experimental import pallas as pl
from jax.experimental.pallas import tpu as pltpu
from jax.experimental.pallas import tpu_sc as plsc

N = 10000
E = 320000
C = 128
R = 8
NB = 4
NR = N * R

NCORES = 2
NSUB = 16
EPT = E // (NCORES * NSUB)
EPA = E // NSUB
CHUNK = 80
SBLK = 2000
SCH = SBLK // CHUNK
ROWS_PER_TILE = 640
ACC_ROWS = NSUB * ROWS_PER_TILE
CNT_PER_TILE = 5120
CNT_TOT = NSUB * CNT_PER_TILE


BR = 1000


def _mk_weight(comp_ref, bases_ref, r):
    wr = comp_ref[r, 0] * bases_ref[0]
    for b in range(1, NB):
        wr = wr + comp_ref[r, b] * bases_ref[b]
    return wr


def _tc1_body(x_ref, comp_ref, bases_ref, root_ref, bias_ref, y_ref, base_ref):
    xb = x_ref[...]
    for r in range(R):
        wr = _mk_weight(comp_ref, bases_ref, r)
        y_ref[:, r * C:(r + 1) * C] = jnp.dot(xb, wr, preferred_element_type=jnp.float32)
    base_ref[...] = jnp.dot(xb, root_ref[...], preferred_element_type=jnp.float32) + bias_ref[...]


def _tc2_body(p_ref, b1_ref, comp_ref, bases_ref, root_ref, bias_ref, y_ref, base_ref):
    h = jnp.maximum(p_ref[0] + p_ref[1] + b1_ref[...], 0.0)
    for r in range(R):
        wr = _mk_weight(comp_ref, bases_ref, r)
        y_ref[:, r * C:(r + 1) * C] = jnp.dot(h, wr, preferred_element_type=jnp.float32)
    base_ref[...] = jnp.dot(h, root_ref[...], preferred_element_type=jnp.float32) + bias_ref[...]


def _tc3_body(p_ref, b2_ref, o_ref):
    o_ref[...] = p_ref[0] + p_ref[1] + b2_ref[...]


_W_SPECS = [
    pl.BlockSpec(memory_space=pltpu.SMEM),
    pl.BlockSpec((NB, C, C), lambda i: (0, 0, 0)),
    pl.BlockSpec((C, C), lambda i: (0, 0)),
    pl.BlockSpec((1, C), lambda i: (0, 0)),
]

_Y_OUT = (
    jax.ShapeDtypeStruct((N, R * C), jnp.float32),
    jax.ShapeDtypeStruct((N, C), jnp.float32),
)
_Y_SPECS = (
    pl.BlockSpec((BR, R * C), lambda i: (i, 0)),
    pl.BlockSpec((BR, C), lambda i: (i, 0)),
)


def _tc1(x, comp, bases, root, bias):
    return pl.pallas_call(
        _tc1_body,
        grid=(N // BR,),
        in_specs=[pl.BlockSpec((BR, C), lambda i: (i, 0))] + _W_SPECS,
        out_specs=_Y_SPECS,
        out_shape=_Y_OUT,
    )(x, comp, bases, root, bias)


def _tc2(part, base1, comp, bases, root, bias):
    return pl.pallas_call(
        _tc2_body,
        grid=(N // BR,),
        in_specs=[
            pl.BlockSpec((NCORES, BR, C), lambda i: (0, i, 0)),
            pl.BlockSpec((BR, C), lambda i: (i, 0)),
        ] + _W_SPECS,
        out_specs=_Y_SPECS,
        out_shape=_Y_OUT,
    )(part, base1, comp, bases, root, bias)


def _tc3(part, base2):
    return pl.pallas_call(
        _tc3_body,
        grid=(N // BR,),
        in_specs=[
            pl.BlockSpec((NCORES, BR, C), lambda i: (0, i, 0)),
            pl.BlockSpec((BR, C), lambda i: (i, 0)),
        ],
        out_specs=pl.BlockSpec((BR, C), lambda i: (i, 0)),
        out_shape=jax.ShapeDtypeStruct((N, C), jnp.float32),
    )(part, base2)


_SC_SCRATCH = [
    pltpu.VMEM_SHARED((ACC_ROWS, C), jnp.float32),
    pltpu.VMEM_SHARED((CNT_TOT,), jnp.float32),
    pltpu.VMEM((SBLK,), jnp.int32),
    pltpu.VMEM((SBLK,), jnp.int32),
    pltpu.VMEM((SBLK,), jnp.int32),
    pltpu.VMEM((CHUNK, C), jnp.float32),
    pltpu.VMEM((CHUNK, C), jnp.float32),
    pltpu.VMEM((CHUNK, C), jnp.float32),
    pltpu.VMEM((CHUNK,), jnp.int32),
    pltpu.VMEM((CHUNK,), jnp.int32),
    pltpu.VMEM((CHUNK,), jnp.int32),
    pltpu.VMEM((CHUNK,), jnp.int32),
    pltpu.VMEM((CHUNK,), jnp.int32),
    pltpu.VMEM((CHUNK,), jnp.int32),
    pltpu.VMEM((SBLK,), jnp.int32),
    pltpu.VMEM((SBLK + 16,), jnp.float32),
    pltpu.VMEM((CHUNK,), jnp.float32),
    pltpu.SemaphoreType.DMA,
    pltpu.SemaphoreType.DMA,
    pltpu.SemaphoreType.DMA,
    pltpu.SemaphoreType.DMA,
    pltpu.SemaphoreType.DMA,
    pltpu.SemaphoreType.DMA,
]


@functools.lru_cache(maxsize=None)
def _sc_mesh():
    return plsc.VectorSubcoreMesh(
        core_axis_name="c", subcore_axis_name="s",
        num_cores=NCORES, num_subcores=NSUB)


def _sc_body(compute_counts, *refs):
    if compute_counts:
        (src_hbm, dst_hbm, et_hbm, y_hbm, zrows_hbm, zcnt_hbm,
         part_hbm, cnt_hbm,
         acc_sh, cnt_sh, srcb, dstb, etb, rows0, rows1, rows2,
         iy0, iy1, iy2, idxd0, idxd1, idxd2, icb, scb, ones,
         gs0, gs1, gs2, ss0, ss1, ss2) = refs
    else:
        (src_hbm, dst_hbm, et_hbm, y_hbm, zrows_hbm, zcnt_hbm, cnt_in_hbm,
         part_hbm,
         acc_sh, cnt_sh, srcb, dstb, etb, rows0, rows1, rows2,
         iy0, iy1, iy2, idxd0, idxd1, idxd2, icb, scb, ones,
         gs0, gs1, gs2, ss0, ss1, ss2) = refs
    gsems = (gs0, gs1, gs2)
    ssems = (ss0, ss1, ss2)
    rowbufs = (rows0, rows1, rows2)
    iybufs = (iy0, iy1, iy2)
    idxdbufs = (idxd0, idxd1, idxd2)
    cid = lax.axis_index("c")
    sid = lax.axis_index("s")

    for q in range(CHUNK // 16):
        ones[pl.ds(q * 16, 16)] = jnp.ones((16,), jnp.float32)

    pltpu.sync_copy(zrows_hbm, acc_sh.at[pl.ds(sid * ROWS_PER_TILE, ROWS_PER_TILE)])

    if compute_counts:
        pltpu.sync_copy(zcnt_hbm, cnt_sh.at[pl.ds(sid * CNT_PER_TILE, CNT_PER_TILE)])
    else:
        pltpu.sync_copy(cnt_in_hbm.at[pl.ds(sid * CNT_PER_TILE, CNT_PER_TILE)],
                        cnt_sh.at[pl.ds(sid * CNT_PER_TILE, CNT_PER_TILE)])

    plsc.subcore_barrier()

    if compute_counts:
        def asuper(sb, _):
            base = sid * EPA + sb * SBLK
            pltpu.sync_copy(dst_hbm.at[pl.ds(base, SBLK)], dstb)
            pltpu.sync_copy(et_hbm.at[pl.ds(base, SBLK)], etb)

            def cbody(k, _):
                for q in range(CHUNK // 16):
                    sl = pl.ds(k * CHUNK + q * 16, 16)
                    idxd0[pl.ds(q * 16, 16)] = dstb[sl] * R + etb[sl]
                pltpu.sync_copy(ones, cnt_sh.at[idxd0], add=True)
                return 0
            lax.fori_loop(0, SCH, cbody, 0)
            return 0
        lax.fori_loop(0, EPA // SBLK, asuper, 0)
        plsc.subcore_barrier()

        @pl.when(cid == 0)
        def _():
            pltpu.sync_copy(cnt_sh.at[pl.ds(sid * CNT_PER_TILE, CNT_PER_TILE)],
                            cnt_hbm.at[pl.ds(sid * CNT_PER_TILE, CNT_PER_TILE)])

    def issue(k, b):
        @pl.when(k >= 3)
        def _():
            pltpu.make_async_copy(rowbufs[b], acc_sh.at[idxdbufs[b]], ssems[b]).wait()
        for q in range(CHUNK // 16):
            sl = pl.ds(k * CHUNK + q * 16, 16)
            iybufs[b][pl.ds(q * 16, 16)] = srcb[sl] * R + etb[sl]
        pltpu.async_copy(y_hbm.at[iybufs[b]], rowbufs[b], gsems[b])

    def process(k, b):
        pltpu.make_async_copy(y_hbm.at[iybufs[b]], rowbufs[b], gsems[b]).wait()
        for q in range(CHUNK // 16):
            dsl = pl.ds(k * CHUNK + q * 16, 16)
            idxdbufs[b][pl.ds(q * 16, 16)] = dstb[dsl]
        rb = rowbufs[b]

        def mgrp(g, _):
            svec = scb[pl.ds(k * CHUNK + g * 16, 16)]
            for l in range(16):
                scj = svec[l]
                row = g * 16 + l
                for q in range(C // 16):
                    sl = pl.ds(q * 16, 16)
                    rb[row, sl] = rb[row, sl] * scj
            return 0
        lax.fori_loop(0, CHUNK // 16, mgrp, 0)
        pltpu.async_copy(rb, acc_sh.at[idxdbufs[b]], ssems[b], add=True)

    def bsuper(sb, _):
        base = sid * EPA + cid * EPT + sb * SBLK
        pltpu.sync_copy(src_hbm.at[pl.ds(base, SBLK)], srcb)
        pltpu.sync_copy(dst_hbm.at[pl.ds(base, SBLK)], dstb)
        pltpu.sync_copy(et_hbm.at[pl.ds(base, SBLK)], etb)

        def icomp(t, _):
            sl = pl.ds(t * 16, 16)
            icb[sl] = dstb[sl] * R + etb[sl]
            return 0
        lax.fori_loop(0, SBLK // 16, icomp, 0)

        def cgath(k2, _):
            pltpu.sync_copy(cnt_sh.at[icb.at[pl.ds(k2 * CHUNK, CHUNK)]],
                            scb.at[pl.ds(k2 * CHUNK, CHUNK)])
            return 0
        lax.fori_loop(0, SCH, cgath, 0)

        def sinv(t, _):
            sl = pl.ds(t * 16, 16)
            scb[sl] = 1.0 / jnp.maximum(scb[sl], 1.0)
            return 0
        lax.fori_loop(0, SBLK // 16, sinv, 0)

        issue(0, 0)
        issue(1, 1)

        def outer(j, _):
            k0 = j * 3
            process(k0, 0)
            issue(k0 + 2, 2)
            process(k0 + 1, 1)
            issue(k0 + 3, 0)
            process(k0 + 2, 2)
            issue(k0 + 4, 1)
            return 0
        lax.fori_loop(0, (SCH - 4) // 3, outer, 0)
        process(21, 0)
        issue(23, 2)
        process(22, 1)
        issue(24, 0)
        process(23, 2)
        process(24, 0)
        pltpu.make_async_copy(rowbufs[1], acc_sh.at[idxdbufs[1]], ssems[1]).wait()
        pltpu.make_async_copy(rowbufs[2], acc_sh.at[idxdbufs[2]], ssems[2]).wait()
        pltpu.make_async_copy(rowbufs[0], acc_sh.at[idxdbufs[0]], ssems[0]).wait()
        return 0
    lax.fori_loop(0, EPT // SBLK, bsuper, 0)

    plsc.subcore_barrier()
    pltpu.sync_copy(acc_sh.at[pl.ds(sid * ROWS_PER_TILE, ROWS_PER_TILE)],
                    part_hbm.at[pl.ds(cid * ACC_ROWS + sid * ROWS_PER_TILE, ROWS_PER_TILE)])


@functools.lru_cache(maxsize=None)
def _sc_pass(compute_counts):
    if compute_counts:
        out_type = (
            jax.ShapeDtypeStruct((NCORES * ACC_ROWS, C), jnp.float32),
            jax.ShapeDtypeStruct((CNT_TOT,), jnp.float32),
        )
    else:
        out_type = jax.ShapeDtypeStruct((NCORES * ACC_ROWS, C), jnp.float32)
    return pl.kernel(
        functools.partial(_sc_body, compute_counts),
        out_type=out_type,
        mesh=_sc_mesh(),
        scratch_types=_SC_SCRATCH,
    )


def kernel(x, edge_index, edge_type, comp1, bases1, root1, bias1,
           comp2, bases2, root2, bias2):
    src = edge_index[0].astype(jnp.int32)
    dst = edge_index[1].astype(jnp.int32)
    et = edge_type.astype(jnp.int32)
    zrows = jnp.zeros((ROWS_PER_TILE, C), jnp.float32)
    zcnt = jnp.zeros((CNT_PER_TILE,), jnp.float32)

    y1, base1 = _tc1(x, comp1, bases1, root1, bias1.reshape(1, C))
    part1, cnt = _sc_pass(True)(src, dst, et, y1.reshape(NR, C), zrows, zcnt)
    y2, base2 = _tc2(part1.reshape(NCORES, ACC_ROWS, C), base1,
                     comp2, bases2, root2, bias2.reshape(1, C))
    part2 = _sc_pass(False)(src, dst, et, y2.reshape(NR, C), zrows, zcnt, cnt)
    return _tc3(part2.reshape(NCORES, ACC_ROWS, C), base2)


# speedup vs baseline: 29.9174x; 1.0109x over previous
import functools

import jax
import jax.numpy as jnp
from jax import lax
from jax.experimental import pallas as pl
from jax.experimental.pallas import tpu as pltpu
from jax.experimental.pallas import tpu_sc as plsc

N = 10000
E = 320000
C = 128
R = 8
NB = 4
NR = N * R

NCORES = 2
NSUB = 16
EPT = E // (NCORES * NSUB)
EPA = E // NSUB
CHUNK = 80
SBLK = 2000
SCH = SBLK // CHUNK
ROWS_PER_TILE = 640
ACC_ROWS = NSUB * ROWS_PER_TILE
CNT_PER_TILE = 5120
CNT_TOT = NSUB * CNT_PER_TILE


BR = 1000


def _mk_weight(comp_ref, bases_ref, r):
    wr = comp_ref[r, 0] * bases_ref[0]
    for b in range(1, NB):
        wr = wr + comp_ref[r, b] * bases_ref[b]
    return wr


def _tc1_body(x_ref, comp_ref, bases_ref, root_ref, bias_ref, y_ref, base_ref):
    xb = x_ref[...]
    for r in range(R):
        wr = _mk_weight(comp_ref, bases_ref, r)
        y_ref[:, r * C:(r + 1) * C] = jnp.dot(xb, wr, preferred_element_type=jnp.float32)
    base_ref[...] = jnp.dot(xb, root_ref[...], preferred_element_type=jnp.float32) + bias_ref[...]


def _tc2_body(p_ref, b1_ref, comp_ref, bases_ref, root_ref, bias_ref, y_ref, base_ref):
    h = jnp.maximum(p_ref[0] + p_ref[1] + b1_ref[...], 0.0)
    for r in range(R):
        wr = _mk_weight(comp_ref, bases_ref, r)
        y_ref[:, r * C:(r + 1) * C] = jnp.dot(h, wr, preferred_element_type=jnp.float32)
    base_ref[...] = jnp.dot(h, root_ref[...], preferred_element_type=jnp.float32) + bias_ref[...]


def _tc3_body(p_ref, b2_ref, o_ref):
    o_ref[...] = p_ref[0] + p_ref[1] + b2_ref[...]


_W_SPECS = [
    pl.BlockSpec(memory_space=pltpu.SMEM),
    pl.BlockSpec((NB, C, C), lambda i: (0, 0, 0)),
    pl.BlockSpec((C, C), lambda i: (0, 0)),
    pl.BlockSpec((1, C), lambda i: (0, 0)),
]

_Y_OUT = (
    jax.ShapeDtypeStruct((N, R * C), jnp.float32),
    jax.ShapeDtypeStruct((N, C), jnp.float32),
)
_Y_SPECS = (
    pl.BlockSpec((BR, R * C), lambda i: (i, 0)),
    pl.BlockSpec((BR, C), lambda i: (i, 0)),
)


def _tc1(x, comp, bases, root, bias):
    return pl.pallas_call(
        _tc1_body,
        grid=(N // BR,),
        in_specs=[pl.BlockSpec((BR, C), lambda i: (i, 0))] + _W_SPECS,
        out_specs=_Y_SPECS,
        out_shape=_Y_OUT,
    )(x, comp, bases, root, bias)


def _tc2(part, base1, comp, bases, root, bias):
    return pl.pallas_call(
        _tc2_body,
        grid=(N // BR,),
        in_specs=[
            pl.BlockSpec((NCORES, BR, C), lambda i: (0, i, 0)),
            pl.BlockSpec((BR, C), lambda i: (i, 0)),
        ] + _W_SPECS,
        out_specs=_Y_SPECS,
        out_shape=_Y_OUT,
    )(part, base1, comp, bases, root, bias)


def _tc3(part, base2):
    return pl.pallas_call(
        _tc3_body,
        grid=(N // BR,),
        in_specs=[
            pl.BlockSpec((NCORES, BR, C), lambda i: (0, i, 0)),
            pl.BlockSpec((BR, C), lambda i: (i, 0)),
        ],
        out_specs=pl.BlockSpec((BR, C), lambda i: (i, 0)),
        out_shape=jax.ShapeDtypeStruct((N, C), jnp.float32),
    )(part, base2)


_SC_CNT_SCRATCH = [
    pltpu.VMEM_SHARED((CNT_TOT,), jnp.float32),
    pltpu.VMEM((SBLK,), jnp.int32),
    pltpu.VMEM((SBLK,), jnp.int32),
    pltpu.VMEM((CHUNK,), jnp.int32),
    pltpu.VMEM((CHUNK,), jnp.float32),
]

_SC_SCRATCH = [
    pltpu.VMEM_SHARED((ACC_ROWS, C), jnp.float32),
    pltpu.VMEM_SHARED((CNT_TOT,), jnp.float32),
    pltpu.VMEM((SBLK,), jnp.int32),
    pltpu.VMEM((SBLK,), jnp.int32),
    pltpu.VMEM((SBLK,), jnp.int32),
    pltpu.VMEM((CHUNK, C), jnp.float32),
    pltpu.VMEM((CHUNK, C), jnp.float32),
    pltpu.VMEM((CHUNK, C), jnp.float32),
    pltpu.VMEM((CHUNK,), jnp.int32),
    pltpu.VMEM((CHUNK,), jnp.int32),
    pltpu.VMEM((CHUNK,), jnp.int32),
    pltpu.VMEM((CHUNK,), jnp.int32),
    pltpu.VMEM((CHUNK,), jnp.int32),
    pltpu.VMEM((CHUNK,), jnp.int32),
    pltpu.VMEM((SBLK,), jnp.int32),
    pltpu.VMEM((SBLK + 16,), jnp.float32),
    pltpu.VMEM((CHUNK,), jnp.float32),
    pltpu.SemaphoreType.DMA,
    pltpu.SemaphoreType.DMA,
    pltpu.SemaphoreType.DMA,
    pltpu.SemaphoreType.DMA,
    pltpu.SemaphoreType.DMA,
    pltpu.SemaphoreType.DMA,
    pltpu.SemaphoreType.DMA,
    pltpu.SemaphoreType.DMA,
]


@functools.lru_cache(maxsize=None)
def _sc_mesh():
    return plsc.VectorSubcoreMesh(
        core_axis_name="c", subcore_axis_name="s",
        num_cores=NCORES, num_subcores=NSUB)


def _sc_cnt_body(dst_hbm, et_hbm, zcnt_hbm, cnt_hbm, cnt_sh, dstb, etb, icc, ones):
    cid = lax.axis_index("c")
    sid = lax.axis_index("s")
    for q in range(CHUNK // 16):
        ones[pl.ds(q * 16, 16)] = jnp.ones((16,), jnp.float32)
    pltpu.sync_copy(zcnt_hbm, cnt_sh.at[pl.ds(sid * CNT_PER_TILE, CNT_PER_TILE)])
    plsc.subcore_barrier()

    def asuper(sb, _):
        base = sid * EPA + sb * SBLK
        pltpu.sync_copy(dst_hbm.at[pl.ds(base, SBLK)], dstb)
        pltpu.sync_copy(et_hbm.at[pl.ds(base, SBLK)], etb)

        def cbody(k, _):
            for q in range(CHUNK // 16):
                sl = pl.ds(k * CHUNK + q * 16, 16)
                icc[pl.ds(q * 16, 16)] = dstb[sl] * R + etb[sl]
            pltpu.sync_copy(ones, cnt_sh.at[icc], add=True)
            return 0
        lax.fori_loop(0, SCH, cbody, 0)
        return 0
    lax.fori_loop(0, EPA // SBLK, asuper, 0)
    plsc.subcore_barrier()

    @pl.when(cid == 0)
    def _():
        pltpu.sync_copy(cnt_sh.at[pl.ds(sid * CNT_PER_TILE, CNT_PER_TILE)],
                        cnt_hbm.at[pl.ds(sid * CNT_PER_TILE, CNT_PER_TILE)])


def _sc_body(src_hbm, dst_hbm, et_hbm, y_hbm, zrows_hbm, cnt_in_hbm, part_hbm,
             acc_sh, cnt_sh, srcb, dstb, etb, rows0, rows1, rows2,
             iy0, iy1, iy2, idxd0, idxd1, idxd2, icb, scb, ones,
             gs0, gs1, gs2, ss0, ss1, ss2, csem, stsem):
    del ones
    gsems = (gs0, gs1, gs2)
    ssems = (ss0, ss1, ss2)
    rowbufs = (rows0, rows1, rows2)
    iybufs = (iy0, iy1, iy2)
    idxdbufs = (idxd0, idxd1, idxd2)
    cid = lax.axis_index("c")
    sid = lax.axis_index("s")

    pltpu.sync_copy(zrows_hbm, acc_sh.at[pl.ds(sid * ROWS_PER_TILE, ROWS_PER_TILE)])
    pltpu.sync_copy(cnt_in_hbm.at[pl.ds(sid * CNT_PER_TILE, CNT_PER_TILE)],
                    cnt_sh.at[pl.ds(sid * CNT_PER_TILE, CNT_PER_TILE)])
    plsc.subcore_barrier()

    def issue(k, b):
        @pl.when(k >= 3)
        def _():
            pltpu.make_async_copy(rowbufs[b], acc_sh.at[idxdbufs[b]], ssems[b]).wait()
        for q in range(CHUNK // 16):
            sl = pl.ds(k * CHUNK + q * 16, 16)
            iybufs[b][pl.ds(q * 16, 16)] = srcb[sl] * R + etb[sl]
            idxdbufs[b][pl.ds(q * 16, 16)] = dstb[sl]
        pltpu.async_copy(y_hbm.at[iybufs[b]], rowbufs[b], gsems[b])

    def process(k, b):
        pltpu.make_async_copy(y_hbm.at[iybufs[b]], rowbufs[b], gsems[b]).wait()
        rb = rowbufs[b]

        @plsc.parallel_loop(0, CHUNK // 16)
        def _(g):
            svec = scb[pl.ds(k * CHUNK + g * 16, 16)]
            for l in range(16):
                scj = svec[l]
                row = g * 16 + l
                for q in range(C // 16):
                    sl = pl.ds(q * 16, 16)
                    rb[row, sl] = rb[row, sl] * scj
        pltpu.async_copy(rb, acc_sh.at[idxdbufs[b]], ssems[b], add=True)

    def _stage_refs(sb):
        base = sid * EPA + cid * EPT + sb * SBLK
        return ((src_hbm.at[pl.ds(base, SBLK)], srcb),
                (dst_hbm.at[pl.ds(base, SBLK)], dstb),
                (et_hbm.at[pl.ds(base, SBLK)], etb))

    def stage_start(sb):
        for hbm_ref, vbuf in _stage_refs(sb):
            pltpu.async_copy(hbm_ref, vbuf, stsem)

    def stage_wait(sb):
        for hbm_ref, vbuf in _stage_refs(sb):
            pltpu.make_async_copy(hbm_ref, vbuf, stsem).wait()

    NSB = EPT // SBLK
    stage_start(0)

    def bsuper(sb, _):
        stage_wait(sb)

        def icomp(k2, _):
            for q in range(CHUNK // 16):
                sl = pl.ds(k2 * CHUNK + q * 16, 16)
                icb[sl] = dstb[sl] * R + etb[sl]
            pltpu.async_copy(cnt_sh.at[icb.at[pl.ds(k2 * CHUNK, CHUNK)]],
                             scb.at[pl.ds(k2 * CHUNK, CHUNK)], csem)
            return 0
        lax.fori_loop(0, SCH, icomp, 0)

        def cdrain(k2, _):
            pltpu.make_async_copy(cnt_sh.at[icb.at[pl.ds(k2 * CHUNK, CHUNK)]],
                                  scb.at[pl.ds(k2 * CHUNK, CHUNK)], csem).wait()
            return 0
        lax.fori_loop(0, SCH, cdrain, 0)

        @plsc.parallel_loop(0, SBLK // 16)
        def _(t):
            sl = pl.ds(t * 16, 16)
            scb[sl] = 1.0 / jnp.maximum(scb[sl], 1.0)

        issue(0, 0)
        issue(1, 1)

        def outer(j, _):
            k0 = j * 3
            process(k0, 0)
            issue(k0 + 2, 2)
            process(k0 + 1, 1)
            issue(k0 + 3, 0)
            process(k0 + 2, 2)
            issue(k0 + 4, 1)
            return 0
        lax.fori_loop(0, (SCH - 4) // 3, outer, 0)
        process(21, 0)
        issue(23, 2)
        process(22, 1)
        issue(24, 0)
        @pl.when(sb < NSB - 1)
        def _():
            stage_start(sb + 1)
        process(23, 2)
        process(24, 0)
        pltpu.make_async_copy(rowbufs[1], acc_sh.at[idxdbufs[1]], ssems[1]).wait()
        pltpu.make_async_copy(rowbufs[2], acc_sh.at[idxdbufs[2]], ssems[2]).wait()
        pltpu.make_async_copy(rowbufs[0], acc_sh.at[idxdbufs[0]], ssems[0]).wait()
        return 0
    lax.fori_loop(0, EPT // SBLK, bsuper, 0)

    plsc.subcore_barrier()
    pltpu.sync_copy(acc_sh.at[pl.ds(sid * ROWS_PER_TILE, ROWS_PER_TILE)],
                    part_hbm.at[pl.ds(cid * ACC_ROWS + sid * ROWS_PER_TILE, ROWS_PER_TILE)])


@functools.lru_cache(maxsize=None)
def _sc_counts():
    return pl.kernel(
        _sc_cnt_body,
        out_type=jax.ShapeDtypeStruct((CNT_TOT,), jnp.float32),
        mesh=_sc_mesh(),
        scratch_types=_SC_CNT_SCRATCH,
    )


@functools.lru_cache(maxsize=None)
def _sc_scatter():
    return pl.kernel(
        _sc_body,
        out_type=jax.ShapeDtypeStruct((NCORES * ACC_ROWS, C), jnp.float32),
        mesh=_sc_mesh(),
        scratch_types=_SC_SCRATCH,
    )


def kernel(x, edge_index, edge_type, comp1, bases1, root1, bias1,
           comp2, bases2, root2, bias2):
    src = edge_index[0].astype(jnp.int32)
    dst = edge_index[1].astype(jnp.int32)
    et = edge_type.astype(jnp.int32)
    zrows = jnp.zeros((ROWS_PER_TILE, C), jnp.float32)
    zcnt = jnp.zeros((CNT_PER_TILE,), jnp.float32)

    cnt = _sc_counts()(dst, et, zcnt)
    y1, base1 = _tc1(x, comp1, bases1, root1, bias1.reshape(1, C))
    part1 = _sc_scatter()(src, dst, et, y1.reshape(NR, C), zrows, cnt)
    y2, base2 = _tc2(part1.reshape(NCORES, ACC_ROWS, C), base1,
                     comp2, bases2, root2, bias2.reshape(1, C))
    part2 = _sc_scatter()(src, dst, et, y2.reshape(NR, C), zrows, cnt)
    return _tc3(part2.reshape(NCORES, ACC_ROWS, C), base2)


# speedup vs baseline: 34.7575x; 1.1618x over previous
import functools

import jax
import jax.numpy as jnp
from jax import lax
from jax.experimental import pallas as pl
from jax.experimental.pallas import tpu as pltpu
from jax.experimental.pallas import tpu_sc as plsc

N = 10000
E = 320000
C = 128
R = 8
NB = 4
NR = N * R

NCORES = 2
NSUB = 16
EPT = E // (NCORES * NSUB)
EPA = E // NSUB
CHUNK = 80
SBLK = 2000
SCH = SBLK // CHUNK
ROWS_PER_TILE = 640
ACC_ROWS = NSUB * ROWS_PER_TILE
CNT_PER_TILE = 5120
CNT_TOT = NSUB * CNT_PER_TILE


BR = 1000


def _mk_weight(comp_ref, bases_ref, r):
    wr = comp_ref[r, 0] * bases_ref[0]
    for b in range(1, NB):
        wr = wr + comp_ref[r, b] * bases_ref[b]
    return wr


def _tc1_body(x_ref, comp_ref, bases_ref, root_ref, bias_ref, y_ref, base_ref):
    xb = x_ref[...]
    for r in range(R):
        wr = _mk_weight(comp_ref, bases_ref, r)
        y_ref[:, r * C:(r + 1) * C] = jnp.dot(xb, wr, preferred_element_type=jnp.float32)
    base_ref[...] = jnp.dot(xb, root_ref[...], preferred_element_type=jnp.float32) + bias_ref[...]


def _tc2_body(p_ref, b1_ref, comp_ref, bases_ref, root_ref, bias_ref, y_ref, base_ref):
    h = jnp.maximum(p_ref[0] + p_ref[1] + b1_ref[...], 0.0)
    for r in range(R):
        wr = _mk_weight(comp_ref, bases_ref, r)
        y_ref[:, r * C:(r + 1) * C] = jnp.dot(h, wr, preferred_element_type=jnp.float32)
    base_ref[...] = jnp.dot(h, root_ref[...], preferred_element_type=jnp.float32) + bias_ref[...]


def _tc3_body(p_ref, b2_ref, o_ref):
    o_ref[...] = p_ref[0] + p_ref[1] + b2_ref[...]


_W_SPECS = [
    pl.BlockSpec(memory_space=pltpu.SMEM),
    pl.BlockSpec((NB, C, C), lambda i: (0, 0, 0)),
    pl.BlockSpec((C, C), lambda i: (0, 0)),
    pl.BlockSpec((1, C), lambda i: (0, 0)),
]

_Y_OUT = (
    jax.ShapeDtypeStruct((N, R * C), jnp.float32),
    jax.ShapeDtypeStruct((N, C), jnp.float32),
)
_Y_SPECS = (
    pl.BlockSpec((BR, R * C), lambda i: (i, 0)),
    pl.BlockSpec((BR, C), lambda i: (i, 0)),
)


def _tc1(x, comp, bases, root, bias):
    return pl.pallas_call(
        _tc1_body,
        grid=(N // BR,),
        in_specs=[pl.BlockSpec((BR, C), lambda i: (i, 0))] + _W_SPECS,
        out_specs=_Y_SPECS,
        out_shape=_Y_OUT,
    )(x, comp, bases, root, bias)


def _tc2(part, base1, comp, bases, root, bias):
    return pl.pallas_call(
        _tc2_body,
        grid=(N // BR,),
        in_specs=[
            pl.BlockSpec((NCORES, BR, C), lambda i: (0, i, 0)),
            pl.BlockSpec((BR, C), lambda i: (i, 0)),
        ] + _W_SPECS,
        out_specs=_Y_SPECS,
        out_shape=_Y_OUT,
    )(part, base1, comp, bases, root, bias)


def _tc3(part, base2):
    return pl.pallas_call(
        _tc3_body,
        grid=(N // BR,),
        in_specs=[
            pl.BlockSpec((NCORES, BR, C), lambda i: (0, i, 0)),
            pl.BlockSpec((BR, C), lambda i: (i, 0)),
        ],
        out_specs=pl.BlockSpec((BR, C), lambda i: (i, 0)),
        out_shape=jax.ShapeDtypeStruct((N, C), jnp.float32),
    )(part, base2)


_SC_CNT_SCRATCH = [
    pltpu.VMEM_SHARED((CNT_TOT,), jnp.float32),
    pltpu.VMEM((SBLK,), jnp.int32),
    pltpu.VMEM((SBLK,), jnp.int32),
    pltpu.VMEM((CHUNK,), jnp.int32),
    pltpu.VMEM((CHUNK,), jnp.float32),
]

_SC_SCRATCH = [
    pltpu.VMEM_SHARED((ACC_ROWS, C), jnp.float32),
    pltpu.VMEM_SHARED((CNT_TOT,), jnp.float32),
    pltpu.VMEM((SBLK,), jnp.int32),
    pltpu.VMEM((SBLK,), jnp.int32),
    pltpu.VMEM((SBLK,), jnp.int32),
    pltpu.VMEM((CHUNK, C), jnp.float32),
    pltpu.VMEM((CHUNK, C), jnp.float32),
    pltpu.VMEM((CHUNK, C), jnp.float32),
    pltpu.VMEM((CHUNK,), jnp.int32),
    pltpu.VMEM((CHUNK,), jnp.int32),
    pltpu.VMEM((CHUNK,), jnp.int32),
    pltpu.VMEM((CHUNK,), jnp.int32),
    pltpu.VMEM((CHUNK,), jnp.int32),
    pltpu.VMEM((CHUNK,), jnp.int32),
    pltpu.VMEM((SBLK,), jnp.int32),
    pltpu.VMEM((SBLK + 16,), jnp.float32),
    pltpu.VMEM((CHUNK,), jnp.float32),
    pltpu.SemaphoreType.DMA,
    pltpu.SemaphoreType.DMA,
    pltpu.SemaphoreType.DMA,
    pltpu.SemaphoreType.DMA,
    pltpu.SemaphoreType.DMA,
    pltpu.SemaphoreType.DMA,
    pltpu.SemaphoreType.DMA,
    pltpu.SemaphoreType.DMA,
]


@functools.lru_cache(maxsize=None)
def _sc_mesh():
    return plsc.VectorSubcoreMesh(
        core_axis_name="c", subcore_axis_name="s",
        num_cores=NCORES, num_subcores=NSUB)


def _sc_cnt_body(dst_hbm, et_hbm, zcnt_hbm, cnt_hbm, cnt_sh, dstb, etb, icc, ones):
    cid = lax.axis_index("c")
    sid = lax.axis_index("s")
    for q in range(CHUNK // 16):
        ones[pl.ds(q * 16, 16)] = jnp.ones((16,), jnp.float32)
    pltpu.sync_copy(zcnt_hbm, cnt_sh.at[pl.ds(sid * CNT_PER_TILE, CNT_PER_TILE)])
    plsc.subcore_barrier()

    def asuper(sb, _):
        base = sid * EPA + sb * SBLK
        pltpu.sync_copy(dst_hbm.at[pl.ds(base, SBLK)], dstb)
        pltpu.sync_copy(et_hbm.at[pl.ds(base, SBLK)], etb)

        def cbody(k, _):
            for q in range(CHUNK // 16):
                sl = pl.ds(k * CHUNK + q * 16, 16)
                icc[pl.ds(q * 16, 16)] = dstb[sl] * R + etb[sl]
            pltpu.sync_copy(ones, cnt_sh.at[icc], add=True)
            return 0
        lax.fori_loop(0, SCH, cbody, 0)
        return 0
    lax.fori_loop(0, EPA // SBLK, asuper, 0)
    plsc.subcore_barrier()

    @pl.when(cid == 0)
    def _():
        pltpu.sync_copy(cnt_sh.at[pl.ds(sid * CNT_PER_TILE, CNT_PER_TILE)],
                        cnt_hbm.at[pl.ds(sid * CNT_PER_TILE, CNT_PER_TILE)])


def _sc_body(src_hbm, dst_hbm, et_hbm, y_hbm, zrows_hbm, cnt_in_hbm, part_hbm,
             acc_sh, cnt_sh, srcb, dstb, etb, rows0, rows1, rows2,
             iy0, iy1, iy2, idxd0, idxd1, idxd2, icb, scb, ones,
             gs0, gs1, gs2, ss0, ss1, ss2, csem, stsem):
    del ones
    gsems = (gs0, gs1, gs2)
    ssems = (ss0, ss1, ss2)
    rowbufs = (rows0, rows1, rows2)
    iybufs = (iy0, iy1, iy2)
    idxdbufs = (idxd0, idxd1, idxd2)
    cid = lax.axis_index("c")
    sid = lax.axis_index("s")

    pltpu.sync_copy(zrows_hbm, acc_sh.at[pl.ds(sid * ROWS_PER_TILE, ROWS_PER_TILE)])
    pltpu.sync_copy(cnt_in_hbm.at[pl.ds(sid * CNT_PER_TILE, CNT_PER_TILE)],
                    cnt_sh.at[pl.ds(sid * CNT_PER_TILE, CNT_PER_TILE)])
    plsc.subcore_barrier()

    def issue(k, b):
        @pl.when(k >= 3)
        def _():
            pltpu.make_async_copy(rowbufs[b], acc_sh.at[idxdbufs[b]], ssems[b]).wait()
        for q in range(CHUNK // 16):
            sl = pl.ds(k * CHUNK + q * 16, 16)
            iybufs[b][pl.ds(q * 16, 16)] = srcb[sl] * R + etb[sl]
            idxdbufs[b][pl.ds(q * 16, 16)] = dstb[sl]
        pltpu.async_copy(y_hbm.at[iybufs[b]], rowbufs[b], gsems[b])

    def process(k, b):
        pltpu.make_async_copy(y_hbm.at[iybufs[b]], rowbufs[b], gsems[b]).wait()
        rb = rowbufs[b]

        def mgrp(g, _):
            svec = scb[pl.ds(k * CHUNK + g * 16, 16)]
            for l in range(16):
                scj = svec[l]
                row = g * 16 + l
                for q in range(C // 16):
                    sl = pl.ds(q * 16, 16)
                    rb[row, sl] = rb[row, sl] * scj
            return 0
        lax.fori_loop(0, CHUNK // 16, mgrp, 0)
        pltpu.async_copy(rb, acc_sh.at[idxdbufs[b]], ssems[b], add=True)

    def _stage_refs(sb):
        base = sid * EPA + cid * EPT + sb * SBLK
        return ((src_hbm.at[pl.ds(base, SBLK)], srcb),
                (dst_hbm.at[pl.ds(base, SBLK)], dstb),
                (et_hbm.at[pl.ds(base, SBLK)], etb))

    def stage_start(sb):
        for hbm_ref, vbuf in _stage_refs(sb):
            pltpu.async_copy(hbm_ref, vbuf, stsem)

    def stage_wait(sb):
        for hbm_ref, vbuf in _stage_refs(sb):
            pltpu.make_async_copy(hbm_ref, vbuf, stsem).wait()

    NSB = EPT // SBLK
    stage_start(0)

    def bsuper(sb, _):
        stage_wait(sb)

        def icomp(k2, _):
            for q in range(CHUNK // 16):
                sl = pl.ds(k2 * CHUNK + q * 16, 16)
                icb[sl] = dstb[sl] * R + etb[sl]
            pltpu.async_copy(cnt_sh.at[icb.at[pl.ds(k2 * CHUNK, CHUNK)]],
                             scb.at[pl.ds(k2 * CHUNK, CHUNK)], csem)
            return 0
        lax.fori_loop(0, SCH, icomp, 0)

        def cdrain(k2, _):
            pltpu.make_async_copy(cnt_sh.at[icb.at[pl.ds(k2 * CHUNK, CHUNK)]],
                                  scb.at[pl.ds(k2 * CHUNK, CHUNK)], csem).wait()
            return 0
        lax.fori_loop(0, SCH, cdrain, 0)

        def sinv(t, _):
            sl = pl.ds(t * 16, 16)
            scb[sl] = 1.0 / jnp.maximum(scb[sl], 1.0)
            return 0
        lax.fori_loop(0, SBLK // 16, sinv, 0)

        issue(0, 0)
        issue(1, 1)

        def outer(j, _):
            k0 = j * 3
            process(k0, 0)
            issue(k0 + 2, 2)
            process(k0 + 1, 1)
            issue(k0 + 3, 0)
            process(k0 + 2, 2)
            issue(k0 + 4, 1)
            return 0
        lax.fori_loop(0, (SCH - 4) // 3, outer, 0)
        process(21, 0)
        issue(23, 2)
        process(22, 1)
        issue(24, 0)
        @pl.when(sb < NSB - 1)
        def _():
            stage_start(sb + 1)
        process(23, 2)
        process(24, 0)
        pltpu.make_async_copy(rowbufs[1], acc_sh.at[idxdbufs[1]], ssems[1]).wait()
        pltpu.make_async_copy(rowbufs[2], acc_sh.at[idxdbufs[2]], ssems[2]).wait()
        pltpu.make_async_copy(rowbufs[0], acc_sh.at[idxdbufs[0]], ssems[0]).wait()
        return 0
    lax.fori_loop(0, EPT // SBLK, bsuper, 0)

    plsc.subcore_barrier()
    pltpu.sync_copy(acc_sh.at[pl.ds(sid * ROWS_PER_TILE, ROWS_PER_TILE)],
                    part_hbm.at[pl.ds(cid * ACC_ROWS + sid * ROWS_PER_TILE, ROWS_PER_TILE)])


@functools.lru_cache(maxsize=None)
def _sc_counts():
    return pl.kernel(
        _sc_cnt_body,
        out_type=jax.ShapeDtypeStruct((CNT_TOT,), jnp.float32),
        mesh=_sc_mesh(),
        scratch_types=_SC_CNT_SCRATCH,
    )


@functools.lru_cache(maxsize=None)
def _sc_scatter():
    return pl.kernel(
        _sc_body,
        out_type=jax.ShapeDtypeStruct((NCORES * ACC_ROWS, C), jnp.float32),
        mesh=_sc_mesh(),
        scratch_types=_SC_SCRATCH,
    )


def kernel(x, edge_index, edge_type, comp1, bases1, root1, bias1,
           comp2, bases2, root2, bias2):
    src = edge_index[0].astype(jnp.int32)
    dst = edge_index[1].astype(jnp.int32)
    et = edge_type.astype(jnp.int32)
    zrows = jnp.zeros((ROWS_PER_TILE, C), jnp.float32)
    zcnt = jnp.zeros((CNT_PER_TILE,), jnp.float32)

    cnt = _sc_counts()(dst, et, zcnt)
    y1, base1 = _tc1(x, comp1, bases1, root1, bias1.reshape(1, C))
    part1 = _sc_scatter()(src, dst, et, y1.reshape(NR, C), zrows, cnt)
    y2, base2 = _tc2(part1.reshape(NCORES, ACC_ROWS, C), base1,
                     comp2, bases2, root2, bias2.reshape(1, C))
    part2 = _sc_scatter()(src, dst, et, y2.reshape(NR, C), zrows, cnt)
    return _tc3(part2.reshape(NCORES, ACC_ROWS, C), base2)
